# baseline (speedup 1.0000x reference)
import functools

import jax
import jax.numpy as jnp
from jax import lax
from jax.experimental import pallas as pl
from jax.experimental.pallas import tpu as pltpu

N = 10000
E = 320000
F_IN = 128
HID = 64
H = 8
C = 8

BLK = 1000
NSTEP = N // BLK
NP = 10240

_INTERPRET = False


def _pc(body, grid, in_specs, out_specs, out_shape, scratch_shapes=()):
    return pl.pallas_call(
        body,
        grid=grid,
        in_specs=in_specs,
        out_specs=out_specs,
        out_shape=out_shape,
        scratch_shapes=list(scratch_shapes),
        interpret=_INTERPRET,
    )


def _attn_tail(hp, att, i, ts_ref, td_ref, m_ref, mx_ref):
    ts_ref[:, 0:64] = hp
    ts_ref[:, 64:72] = att[:, 0:8]
    ts_ref[:, 72:80] = jnp.zeros((BLK, 8), jnp.float32)
    td_ref[:, 0:8] = att[:, 8:16]
    td_ref[:, 8:16] = jnp.zeros((BLK, 8), jnp.float32)
    bmax = jnp.max(att, axis=0, keepdims=True)

    @pl.when(i == 0)
    def _():
        mx_ref[...] = bmax

    @pl.when(i > 0)
    def _():
        mx_ref[...] = jnp.maximum(mx_ref[...], bmax)

    @pl.when(i == NSTEP - 1)
    def _():
        s = mx_ref[0, 0:8] + mx_ref[0, 8:16]
        m_ref[0, 0:8] = jnp.maximum(s, 0.2 * s)
        m_ref[0, 8:16] = jnp.full((8,), 1e30, jnp.float32)


def _prelude_body(x_ref, w_ref, b_ref, wg_ref, asd_ref, ts_ref, td_ref, m_ref,
                  mx_ref):
    i = pl.program_id(0)
    h = jnp.dot(x_ref[...], w_ref[...], preferred_element_type=jnp.float32)
    h = jnp.maximum(h + b_ref[...], 0.0)
    hp = jnp.dot(h, wg_ref[...], preferred_element_type=jnp.float32)
    att = jnp.dot(hp, asd_ref[...], preferred_element_type=jnp.float32)
    _attn_tail(hp, att, i, ts_ref, td_ref, m_ref, mx_ref)


def _merge_body(a0_ref, a1_ref, bg_ref, r_ref, x1_ref, st_ref, s1_ref, s2_ref):
    i = pl.program_id(0)
    s = a0_ref[...] + a1_ref[...]
    inv = 1.0 / s[:, 64:72]
    invb = jnp.dot(inv, r_ref[...], preferred_element_type=jnp.float32)
    x1 = s[:, 0:64] * invb + bg_ref[...]
    x1_ref[...] = x1
    ps = jnp.sum(x1, axis=0, keepdims=True)
    ps2 = jnp.sum(x1 * x1, axis=0, keepdims=True)

    @pl.when(i == 0)
    def _():
        s1_ref[...] = ps
        s2_ref[...] = ps2

    @pl.when(i > 0)
    def _():
        s1_ref[...] += ps
        s2_ref[...] += ps2

    @pl.when(i == NSTEP - 1)
    def _():
        mean = s1_ref[...] / N
        st_ref[0:1, :] = mean
        st_ref[1:2, :] = s2_ref[...] / N - mean * mean


def _bn_relu(x1, st_ref, g_ref, be_ref):
    xn = (x1 - st_ref[0:1, :]) * lax.rsqrt(st_ref[1:2, :] + 1e-5)
    return jnp.maximum(xn * g_ref[...] + be_ref[...], 0.0)


def _transform_body(x1_ref, st_ref, g_ref, be_ref, wg_ref, asd_ref, ts_ref,
                    td_ref, m_ref, mx_ref):
    i = pl.program_id(0)
    h = _bn_relu(x1_ref[...], st_ref, g_ref, be_ref)
    hp = jnp.dot(h, wg_ref[...], preferred_element_type=jnp.float32)
    att = jnp.dot(hp, asd_ref[...], preferred_element_type=jnp.float32)
    _attn_tail(hp, att, i, ts_ref, td_ref, m_ref, mx_ref)


def _final_body(x1_ref, st_ref, g_ref, be_ref, w1_ref, b1_ref, w2_ref, b2_ref,
                o_ref, ps_ref):
    i = pl.program_id(0)
    h = _bn_relu(x1_ref[...], st_ref, g_ref, be_ref)
    ps = jnp.sum(h, axis=0, keepdims=True)

    @pl.when(i == 0)
    def _():
        ps_ref[...] = ps

    @pl.when(i > 0)
    def _():
        ps_ref[...] += ps

    @pl.when(i == NSTEP - 1)
    def _():
        pooled = ps_ref[...] / N
        z = jnp.dot(pooled, w1_ref[...], preferred_element_type=jnp.float32)
        z = jnp.maximum(z + b1_ref[...], 0.0)
        o = jnp.dot(z, w2_ref[...], preferred_element_type=jnp.float32)
        o_ref[...] = o + b2_ref[...]


_f32 = jnp.float32


def _prelude(x, w_in_t, b_in, wg_t, asd):
    return _pc(
        _prelude_body,
        grid=(NSTEP,),
        in_specs=[
            pl.BlockSpec((BLK, F_IN), lambda i: (i, 0)),
            pl.BlockSpec((F_IN, HID), lambda i: (0, 0)),
            pl.BlockSpec((1, HID), lambda i: (0, 0)),
            pl.BlockSpec((HID, HID), lambda i: (0, 0)),
            pl.BlockSpec((HID, 16), lambda i: (0, 0)),
        ],
        out_specs=[
            pl.BlockSpec((BLK, 80), lambda i: (i, 0)),
            pl.BlockSpec((BLK, 16), lambda i: (i, 0)),
            pl.BlockSpec((1, 16), lambda i: (0, 0)),
        ],
        out_shape=[
            jax.ShapeDtypeStruct((N, 80), _f32),
            jax.ShapeDtypeStruct((N, 16), _f32),
            jax.ShapeDtypeStruct((1, 16), _f32),
        ],
        scratch_shapes=[pltpu.VMEM((1, 16), _f32)],
    )(x, w_in_t, b_in, wg_t, asd)


def _merge(acc0, acc1, bg, rmat):
    return _pc(
        _merge_body,
        grid=(NSTEP,),
        in_specs=[
            pl.BlockSpec((BLK, 80), lambda i: (i, 0)),
            pl.BlockSpec((BLK, 80), lambda i: (i, 0)),
            pl.BlockSpec((1, HID), lambda i: (0, 0)),
            pl.BlockSpec((8, HID), lambda i: (0, 0)),
        ],
        out_specs=[
            pl.BlockSpec((BLK, HID), lambda i: (i, 0)),
            pl.BlockSpec((2, HID), lambda i: (0, 0)),
        ],
        out_shape=[
            jax.ShapeDtypeStruct((N, HID), _f32),
            jax.ShapeDtypeStruct((2, HID), _f32),
        ],
        scratch_shapes=[pltpu.VMEM((1, HID), _f32), pltpu.VMEM((1, HID), _f32)],
    )(acc0, acc1, bg, rmat)


def _transform(x1, st, g, be, wg_t, asd):
    return _pc(
        _transform_body,
        grid=(NSTEP,),
        in_specs=[
            pl.BlockSpec((BLK, HID), lambda i: (i, 0)),
            pl.BlockSpec((2, HID), lambda i: (0, 0)),
            pl.BlockSpec((1, HID), lambda i: (0, 0)),
            pl.BlockSpec((1, HID), lambda i: (0, 0)),
            pl.BlockSpec((HID, HID), lambda i: (0, 0)),
            pl.BlockSpec((HID, 16), lambda i: (0, 0)),
        ],
        out_specs=[
            pl.BlockSpec((BLK, 80), lambda i: (i, 0)),
            pl.BlockSpec((BLK, 16), lambda i: (i, 0)),
            pl.BlockSpec((1, 16), lambda i: (0, 0)),
        ],
        out_shape=[
            jax.ShapeDtypeStruct((N, 80), _f32),
            jax.ShapeDtypeStruct((N, 16), _f32),
            jax.ShapeDtypeStruct((1, 16), _f32),
        ],
        scratch_shapes=[pltpu.VMEM((1, 16), _f32)],
    )(x1, st, g, be, wg_t, asd)


def _final(x1, st, g, be, w1_t, b1, w2_t, b2):
    return _pc(
        _final_body,
        grid=(NSTEP,),
        in_specs=[
            pl.BlockSpec((BLK, HID), lambda i: (i, 0)),
            pl.BlockSpec((2, HID), lambda i: (0, 0)),
            pl.BlockSpec((1, HID), lambda i: (0, 0)),
            pl.BlockSpec((1, HID), lambda i: (0, 0)),
            pl.BlockSpec((HID, HID // 2), lambda i: (0, 0)),
            pl.BlockSpec((1, HID // 2), lambda i: (0, 0)),
            pl.BlockSpec((HID // 2, 2), lambda i: (0, 0)),
            pl.BlockSpec((1, 2), lambda i: (0, 0)),
        ],
        out_specs=pl.BlockSpec((1, 2), lambda i: (0, 0)),
        out_shape=jax.ShapeDtypeStruct((1, 2), _f32),
        scratch_shapes=[pltpu.VMEM((1, HID), _f32)],
    )(x1, st, g, be, w1_t, b1, w2_t, b2)


def _edge_xla(t_src, t_dst, m16, src, dst):
    asrc = t_src[src, 64:72]
    adst = t_dst[dst, 0:8]
    a = asrc + adst
    a = jnp.maximum(a, 0.2 * a)
    ex = jnp.exp(a - m16[0, 0:8])
    hp = t_src[src, 0:64]
    num = jax.ops.segment_sum(hp * jnp.repeat(ex, 8, axis=1), dst,
                              num_segments=N)
    den = jax.ops.segment_sum(ex, dst, num_segments=N)
    acc0 = jnp.zeros((NP, 80), _f32)
    acc0 = acc0.at[:N, 0:64].set(num).at[:N, 64:72].set(den)
    return acc0, jnp.zeros((NP, 80), _f32)


def _asd(a_s, a_d):
    rows = jnp.arange(64)
    heads = rows // 8
    a = jnp.zeros((64, 16), _f32)
    a = a.at[rows, heads].set(a_s.reshape(64).astype(_f32))
    a = a.at[rows, 8 + heads].set(a_d.reshape(64).astype(_f32))
    return a


def kernel(x, edge_index, W_in, b_in, Wg0, as0, ad0, bg0, g0, be0, Wg1, as1,
           ad1, bg1, g1, be1, Wg2, as2, ad2, bg2, g2, be2, W_fc1, b_fc1,
           W_fc2, b_fc2):
    loop = jnp.arange(N, dtype=edge_index.dtype)
    src = jnp.concatenate([edge_index[0], loop])
    dst = jnp.concatenate([edge_index[1], loop])

    rmat = jnp.repeat(jnp.eye(8, dtype=_f32), 8, axis=1)
    asds = [_asd(as0, ad0), _asd(as1, ad1), _asd(as2, ad2)]
    wgs = [Wg0.T, Wg1.T, Wg2.T]
    bgs = [bg0[None, :], bg1[None, :], bg2[None, :]]
    gs = [g0[None, :], g1[None, :], g2[None, :]]
    bes = [be0[None, :], be1[None, :], be2[None, :]]

    ts, tdm, m16 = _prelude(x, W_in.T, b_in[None, :], wgs[0], asds[0])
    for layer in range(3):
        td = jnp.concatenate([tdm, jnp.zeros((1, 16), _f32)], axis=0)
        acc0, acc1 = _edge_xla(ts, td, m16, src, dst)
        x1, st = _merge(acc0, acc1, bgs[layer], rmat)
        if layer < 2:
            ts, tdm, m16 = _transform(x1, st, gs[layer], bes[layer],
                                      wgs[layer + 1], asds[layer + 1])
        else:
            out = _final(x1, st, gs[layer], bes[layer], W_fc1.T,
                         b_fc1[None, :], W_fc2.T, b_fc2[None, :])
    return out


# speedup vs baseline: 134.2487x; 134.2487x over previous
import functools

import jax
import jax.numpy as jnp
from jax import lax
from jax.experimental import pallas as pl
from jax.experimental.pallas import tpu as pltpu
from jax.experimental.pallas import tpu_sc as plsc

N = 10000
E = 320000
F_IN = 128
HID = 64

BLK = 2000
NSTEP = N // BLK
NP = 10240

K_E = 64
NWORK = 32
NCHUNK = -(-(E + N) // (NWORK * K_E))
EPAD = NWORK * K_E * NCHUNK
ROWS_PER_SUB = NP // 16

_INTERPRET = False
_f32 = jnp.float32
_i32 = jnp.int32


def _pc(body, grid, in_specs, out_specs, out_shape, scratch_shapes=()):
    return pl.pallas_call(
        body,
        grid=grid,
        in_specs=in_specs,
        out_specs=out_specs,
        out_shape=out_shape,
        scratch_shapes=list(scratch_shapes),
        interpret=_INTERPRET,
    )


def _attn_tail(hp, att, i, ta_ref, m_ref, mx_ref):
    ta_ref[:, 0:64] = hp
    ta_ref[:, 64:72] = att[:, 0:8]
    ta_ref[:, 72:80] = att[:, 8:16]
    ta_ref[:, 80:128] = jnp.zeros((BLK, 48), _f32)
    bmax = jnp.max(att, axis=0, keepdims=True)

    @pl.when(i == 0)
    def _():
        mx_ref[...] = bmax

    @pl.when(i > 0)
    def _():
        mx_ref[...] = jnp.maximum(mx_ref[...], bmax)

    @pl.when(i == NSTEP - 1)
    def _():
        s = mx_ref[0, 0:8] + mx_ref[0, 8:16]
        m_ref[0, 0:8] = jnp.maximum(s, 0.2 * s)
        m_ref[0, 8:16] = jnp.full((8,), 1e30, _f32)


def _prelude_body(x_ref, w_ref, b_ref, wg_ref, asd_ref, ta_ref, m_ref,
                  mx_ref):
    i = pl.program_id(0)
    h = jnp.dot(x_ref[...], w_ref[...], preferred_element_type=_f32)
    h = jnp.maximum(h + b_ref[...], 0.0)
    hp = jnp.dot(h, wg_ref[...], preferred_element_type=_f32)
    att = jnp.dot(hp, asd_ref[...], preferred_element_type=_f32)
    _attn_tail(hp, att, i, ta_ref, m_ref, mx_ref)


def _merge_body(a0_ref, a1_ref, bg_ref, r_ref, x1_ref, st_ref, s1_ref, s2_ref):
    i = pl.program_id(0)
    s = a0_ref[...] + a1_ref[...]
    inv = 1.0 / jnp.maximum(s[:, 64:72], 1e-30)
    invb = jnp.dot(inv, r_ref[...], preferred_element_type=_f32)
    x1 = s[:, 0:64] * invb + bg_ref[...]
    x1_ref[...] = x1
    ps = jnp.sum(x1, axis=0, keepdims=True)
    ps2 = jnp.sum(x1 * x1, axis=0, keepdims=True)

    @pl.when(i == 0)
    def _():
        s1_ref[...] = ps
        s2_ref[...] = ps2

    @pl.when(i > 0)
    def _():
        s1_ref[...] += ps
        s2_ref[...] += ps2

    @pl.when(i == NSTEP - 1)
    def _():
        mean = s1_ref[...] / N
        st_ref[0:1, :] = mean
        st_ref[1:2, :] = s2_ref[...] / N - mean * mean


def _bn_relu(x1, st_ref, g_ref, be_ref):
    xn = (x1 - st_ref[0:1, :]) * lax.rsqrt(st_ref[1:2, :] + 1e-5)
    return jnp.maximum(xn * g_ref[...] + be_ref[...], 0.0)


def _transform_body(x1_ref, st_ref, g_ref, be_ref, wg_ref, asd_ref, ta_ref,
                    m_ref, mx_ref):
    i = pl.program_id(0)
    h = _bn_relu(x1_ref[...], st_ref, g_ref, be_ref)
    hp = jnp.dot(h, wg_ref[...], preferred_element_type=_f32)
    att = jnp.dot(hp, asd_ref[...], preferred_element_type=_f32)
    _attn_tail(hp, att, i, ta_ref, m_ref, mx_ref)


def _final_body(x1_ref, st_ref, g_ref, be_ref, w1_ref, b1_ref, w2_ref, b2_ref,
                o_ref, ps_ref):
    i = pl.program_id(0)
    h = _bn_relu(x1_ref[...], st_ref, g_ref, be_ref)
    ps = jnp.sum(h, axis=0, keepdims=True)

    @pl.when(i == 0)
    def _():
        ps_ref[...] = ps

    @pl.when(i > 0)
    def _():
        ps_ref[...] += ps

    @pl.when(i == NSTEP - 1)
    def _():
        pooled = ps_ref[...] / N
        z = jnp.dot(pooled, w1_ref[...], preferred_element_type=_f32)
        z = jnp.maximum(z + b1_ref[...], 0.0)
        o = jnp.dot(z, w2_ref[...], preferred_element_type=_f32)
        o_ref[...] = o + b2_ref[...]


_TAB_SPECS = [
    pl.BlockSpec((BLK, 128), lambda i: (i, 0)),
    pl.BlockSpec((1, 16), lambda i: (0, 0)),
]
_TAB_SHAPES = [
    jax.ShapeDtypeStruct((N, 128), _f32),
    jax.ShapeDtypeStruct((1, 16), _f32),
]


def _prelude(x, w_in_t, b_in, wg_t, asd):
    return _pc(
        _prelude_body,
        grid=(NSTEP,),
        in_specs=[
            pl.BlockSpec((BLK, F_IN), lambda i: (i, 0)),
            pl.BlockSpec((F_IN, HID), lambda i: (0, 0)),
            pl.BlockSpec((1, HID), lambda i: (0, 0)),
            pl.BlockSpec((HID, HID), lambda i: (0, 0)),
            pl.BlockSpec((HID, 16), lambda i: (0, 0)),
        ],
        out_specs=_TAB_SPECS,
        out_shape=_TAB_SHAPES,
        scratch_shapes=[pltpu.VMEM((1, 16), _f32)],
    )(x, w_in_t, b_in, wg_t, asd)


def _merge(acc0, acc1, bg, rmat):
    return _pc(
        _merge_body,
        grid=(NSTEP,),
        in_specs=[
            pl.BlockSpec((BLK, 128), lambda i: (i, 0)),
            pl.BlockSpec((BLK, 128), lambda i: (i, 0)),
            pl.BlockSpec((1, HID), lambda i: (0, 0)),
            pl.BlockSpec((8, HID), lambda i: (0, 0)),
        ],
        out_specs=[
            pl.BlockSpec((BLK, HID), lambda i: (i, 0)),
            pl.BlockSpec((2, HID), lambda i: (0, 0)),
        ],
        out_shape=[
            jax.ShapeDtypeStruct((N, HID), _f32),
            jax.ShapeDtypeStruct((2, HID), _f32),
        ],
        scratch_shapes=[pltpu.VMEM((1, HID), _f32), pltpu.VMEM((1, HID), _f32)],
    )(acc0, acc1, bg, rmat)


def _transform(x1, st, g, be, wg_t, asd):
    return _pc(
        _transform_body,
        grid=(NSTEP,),
        in_specs=[
            pl.BlockSpec((BLK, HID), lambda i: (i, 0)),
            pl.BlockSpec((2, HID), lambda i: (0, 0)),
            pl.BlockSpec((1, HID), lambda i: (0, 0)),
            pl.BlockSpec((1, HID), lambda i: (0, 0)),
            pl.BlockSpec((HID, HID), lambda i: (0, 0)),
            pl.BlockSpec((HID, 16), lambda i: (0, 0)),
        ],
        out_specs=_TAB_SPECS,
        out_shape=_TAB_SHAPES,
        scratch_shapes=[pltpu.VMEM((1, 16), _f32)],
    )(x1, st, g, be, wg_t, asd)


def _final(x1, st, g, be, w1_t, b1, w2_t, b2):
    return _pc(
        _final_body,
        grid=(NSTEP,),
        in_specs=[
            pl.BlockSpec((BLK, HID), lambda i: (i, 0)),
            pl.BlockSpec((2, HID), lambda i: (0, 0)),
            pl.BlockSpec((1, HID), lambda i: (0, 0)),
            pl.BlockSpec((1, HID), lambda i: (0, 0)),
            pl.BlockSpec((HID, HID // 2), lambda i: (0, 0)),
            pl.BlockSpec((1, HID // 2), lambda i: (0, 0)),
            pl.BlockSpec((HID // 2, 2), lambda i: (0, 0)),
            pl.BlockSpec((1, 2), lambda i: (0, 0)),
        ],
        out_specs=pl.BlockSpec((1, 2), lambda i: (0, 0)),
        out_shape=jax.ShapeDtypeStruct((1, 2), _f32),
        scratch_shapes=[pltpu.VMEM((1, HID), _f32)],
    )(x1, st, g, be, w1_t, b1, w2_t, b2)


def _dyn_gather16(x, idx):
    dn = lax.GatherDimensionNumbers(offset_dims=(), collapsed_slice_dims=(0,),
                                    start_index_map=(0,))
    return lax.gather(x, idx.reshape(16, 1), dn, (1,),
                      mode=lax.GatherScatterMode.PROMISE_IN_BOUNDS)


def _edge_sc(t_all, m16, csd, dsts):
    mesh = plsc.VectorSubcoreMesh(core_axis_name="c", subcore_axis_name="s")

    @functools.partial(
        pl.kernel,
        out_type=jax.ShapeDtypeStruct((2 * NP, 128), _f32),
        mesh=mesh,
        scratch_types=[
            pltpu.VMEM((2 * K_E,), _i32),
            pltpu.VMEM((K_E,), _i32),
            pltpu.VMEM((2 * K_E, 128), _f32),
            pltpu.VMEM((K_E, 128), _f32),
            pltpu.VMEM((16,), _f32),
            pltpu.VMEM_SHARED((NP, 128), _f32),
        ],
    )
    def k(ta_hbm, m_hbm, csd_hbm, dsts_hbm, out_hbm,
          cidx, didxs, rows, stage, m_v, acc):
        cid = lax.axis_index("c")
        sid = lax.axis_index("s")
        wid = sid * 2 + cid
        row0 = sid * ROWS_PER_SUB

        @pl.loop(0, K_E)
        def _(r):
            @pl.loop(0, 128, step=16)
            def _(cc):
                stage[r, pl.ds(cc, 16)] = jnp.zeros((16,), _f32)

        @pl.loop(0, ROWS_PER_SUB, step=K_E)
        def _(rr):
            pltpu.sync_copy(stage, acc.at[pl.ds(row0 + rr, K_E)])

        pltpu.sync_copy(m_hbm, m_v)
        plsc.subcore_barrier()

        iota = lax.iota(_i32, 16)
        ib = lax.shift_right_logical(iota, 3)
        rot8 = lax.bitwise_and(iota + 8, 15)

        @pl.loop(0, NCHUNK)
        def _(ck):
            k2 = wid * NCHUNK + ck
            pltpu.sync_copy(csd_hbm.at[pl.ds(k2 * 2 * K_E, 2 * K_E)], cidx)
            pltpu.sync_copy(dsts_hbm.at[pl.ds(k2 * K_E, K_E)], didxs)
            pltpu.sync_copy(ta_hbm.at[cidx], rows)
            mv = m_v[...]

            @pl.loop(0, K_E)
            def _(e):
                asr = rows[e, pl.ds(64, 16)]
                ads = _dyn_gather16(rows[K_E + e, pl.ds(64, 16)], rot8)
                a = asr + ads
                a = jnp.maximum(a, 0.2 * a)
                ex = jnp.exp(a - mv)
                stage[e, pl.ds(64, 16)] = ex
                for g in range(4):
                    exb = _dyn_gather16(ex, ib + (2 * g))
                    stage[e, pl.ds(16 * g, 16)] = (
                        exb * rows[e, pl.ds(16 * g, 16)])

            pltpu.sync_copy(stage, acc.at[didxs], add=True)

        plsc.subcore_barrier()
        pltpu.sync_copy(acc.at[pl.ds(row0, ROWS_PER_SUB)],
                        out_hbm.at[pl.ds(cid * NP + row0, ROWS_PER_SUB)])

    accs = k(t_all, m16, csd, dsts)
    return accs[:NP], accs[NP:]


def _asd(a_s, a_d):
    rows = jnp.arange(64)
    heads = rows // 8
    a = jnp.zeros((64, 16), _f32)
    a = a.at[rows, heads].set(a_s.reshape(64).astype(_f32))
    a = a.at[rows, 8 + heads].set(a_d.reshape(64).astype(_f32))
    return a


def kernel(x, edge_index, W_in, b_in, Wg0, as0, ad0, bg0, g0, be0, Wg1, as1,
           ad1, bg1, g1, be1, Wg2, as2, ad2, bg2, g2, be2, W_fc1, b_fc1,
           W_fc2, b_fc2):
    loop = jnp.arange(N, dtype=edge_index.dtype)
    npad = EPAD - (E + N)
    src = jnp.concatenate(
        [edge_index[0], loop, jnp.zeros((npad,), edge_index.dtype)])
    dstg = jnp.concatenate(
        [edge_index[1], loop, jnp.zeros((npad,), edge_index.dtype)])
    dsts = jnp.concatenate(
        [edge_index[1], loop, jnp.full((npad,), N, edge_index.dtype)])
    csd = jnp.concatenate(
        [src.reshape(-1, K_E), dstg.reshape(-1, K_E)], axis=1).reshape(-1)

    rmat = jnp.repeat(jnp.eye(8, dtype=_f32), 8, axis=1)
    asds = [_asd(as0, ad0), _asd(as1, ad1), _asd(as2, ad2)]
    wgs = [Wg0.T, Wg1.T, Wg2.T]
    bgs = [bg0[None, :], bg1[None, :], bg2[None, :]]
    gs = [g0[None, :], g1[None, :], g2[None, :]]
    bes = [be0[None, :], be1[None, :], be2[None, :]]

    ta, m16 = _prelude(x, W_in.T, b_in[None, :], wgs[0], asds[0])
    for layer in range(3):
        acc0, acc1 = _edge_sc(ta, m16.reshape(16), csd, dsts)
        x1, st = _merge(acc0, acc1, bgs[layer], rmat)
        if layer < 2:
            ta, m16 = _transform(x1, st, gs[layer], bes[layer],
                                 wgs[layer + 1], asds[layer + 1])
        else:
            out = _final(x1, st, gs[layer], bes[layer], W_fc1.T,
                         b_fc1[None, :], W_fc2.T, b_fc2[None, :])
    return out


# speedup vs baseline: 156.1082x; 1.1628x over previous
import functools

import jax
import jax.numpy as jnp
from jax import lax
from jax.experimental import pallas as pl
from jax.experimental.pallas import tpu as pltpu
from jax.experimental.pallas import tpu_sc as plsc

N = 10000
E = 320000
F_IN = 128
HID = 64

BLK = 2000
NSTEP = N // BLK
NP = 10240

K_E = 48
NWORK = 32
NCHUNK = 2 * (-(-(E + N) // (2 * NWORK * K_E)))
EPAD = NWORK * K_E * NCHUNK
ROWS_PER_SUB = NP // 16

_INTERPRET = False
_f32 = jnp.float32
_i32 = jnp.int32


def _pc(body, grid, in_specs, out_specs, out_shape, scratch_shapes=()):
    return pl.pallas_call(
        body,
        grid=grid,
        in_specs=in_specs,
        out_specs=out_specs,
        out_shape=out_shape,
        scratch_shapes=list(scratch_shapes),
        interpret=_INTERPRET,
    )


def _attn_tail(hp, att, i, ta_ref, m_ref, mx_ref):
    ta_ref[:, 0:64] = hp
    ta_ref[:, 64:72] = att[:, 0:8]
    ta_ref[:, 72:80] = att[:, 8:16]
    ta_ref[:, 80:128] = jnp.zeros((BLK, 48), _f32)
    bmax = jnp.max(att, axis=0, keepdims=True)

    @pl.when(i == 0)
    def _():
        mx_ref[...] = bmax

    @pl.when(i > 0)
    def _():
        mx_ref[...] = jnp.maximum(mx_ref[...], bmax)

    @pl.when(i == NSTEP - 1)
    def _():
        s = mx_ref[0, 0:8] + mx_ref[0, 8:16]
        m_ref[0, 0:8] = jnp.maximum(s, 0.2 * s)
        m_ref[0, 8:16] = jnp.full((8,), 1e30, _f32)


def _prelude_body(x_ref, w_ref, b_ref, wg_ref, asd_ref, ta_ref, m_ref,
                  mx_ref):
    i = pl.program_id(0)
    h = jnp.dot(x_ref[...], w_ref[...], preferred_element_type=_f32)
    h = jnp.maximum(h + b_ref[...], 0.0)
    hp = jnp.dot(h, wg_ref[...], preferred_element_type=_f32)
    att = jnp.dot(hp, asd_ref[...], preferred_element_type=_f32)
    _attn_tail(hp, att, i, ta_ref, m_ref, mx_ref)


def _merge_body(a0_ref, a1_ref, bg_ref, r_ref, x1_ref, st_ref, s1_ref, s2_ref):
    i = pl.program_id(0)
    s = a0_ref[...] + a1_ref[...]
    inv = 1.0 / jnp.maximum(s[:, 64:72], 1e-30)
    invb = jnp.dot(inv, r_ref[...], preferred_element_type=_f32)
    x1 = s[:, 0:64] * invb + bg_ref[...]
    x1_ref[...] = x1
    ps = jnp.sum(x1, axis=0, keepdims=True)
    ps2 = jnp.sum(x1 * x1, axis=0, keepdims=True)

    @pl.when(i == 0)
    def _():
        s1_ref[...] = ps
        s2_ref[...] = ps2

    @pl.when(i > 0)
    def _():
        s1_ref[...] += ps
        s2_ref[...] += ps2

    @pl.when(i == NSTEP - 1)
    def _():
        mean = s1_ref[...] / N
        st_ref[0:1, :] = mean
        st_ref[1:2, :] = s2_ref[...] / N - mean * mean


def _bn_relu(x1, st_ref, g_ref, be_ref):
    xn = (x1 - st_ref[0:1, :]) * lax.rsqrt(st_ref[1:2, :] + 1e-5)
    return jnp.maximum(xn * g_ref[...] + be_ref[...], 0.0)


def _transform_body(x1_ref, st_ref, g_ref, be_ref, wg_ref, asd_ref, ta_ref,
                    m_ref, mx_ref):
    i = pl.program_id(0)
    h = _bn_relu(x1_ref[...], st_ref, g_ref, be_ref)
    hp = jnp.dot(h, wg_ref[...], preferred_element_type=_f32)
    att = jnp.dot(hp, asd_ref[...], preferred_element_type=_f32)
    _attn_tail(hp, att, i, ta_ref, m_ref, mx_ref)


def _final_body(x1_ref, st_ref, g_ref, be_ref, w1_ref, b1_ref, w2_ref, b2_ref,
                o_ref, ps_ref):
    i = pl.program_id(0)
    h = _bn_relu(x1_ref[...], st_ref, g_ref, be_ref)
    ps = jnp.sum(h, axis=0, keepdims=True)

    @pl.when(i == 0)
    def _():
        ps_ref[...] = ps

    @pl.when(i > 0)
    def _():
        ps_ref[...] += ps

    @pl.when(i == NSTEP - 1)
    def _():
        pooled = ps_ref[...] / N
        z = jnp.dot(pooled, w1_ref[...], preferred_element_type=_f32)
        z = jnp.maximum(z + b1_ref[...], 0.0)
        o = jnp.dot(z, w2_ref[...], preferred_element_type=_f32)
        o_ref[...] = o + b2_ref[...]


_TAB_SPECS = [
    pl.BlockSpec((BLK, 128), lambda i: (i, 0)),
    pl.BlockSpec((1, 16), lambda i: (0, 0)),
]
_TAB_SHAPES = [
    jax.ShapeDtypeStruct((N, 128), _f32),
    jax.ShapeDtypeStruct((1, 16), _f32),
]


def _prelude(x, w_in_t, b_in, wg_t, asd):
    return _pc(
        _prelude_body,
        grid=(NSTEP,),
        in_specs=[
            pl.BlockSpec((BLK, F_IN), lambda i: (i, 0)),
            pl.BlockSpec((F_IN, HID), lambda i: (0, 0)),
            pl.BlockSpec((1, HID), lambda i: (0, 0)),
            pl.BlockSpec((HID, HID), lambda i: (0, 0)),
            pl.BlockSpec((HID, 16), lambda i: (0, 0)),
        ],
        out_specs=_TAB_SPECS,
        out_shape=_TAB_SHAPES,
        scratch_shapes=[pltpu.VMEM((1, 16), _f32)],
    )(x, w_in_t, b_in, wg_t, asd)


def _merge(acc0, acc1, bg, rmat):
    return _pc(
        _merge_body,
        grid=(NSTEP,),
        in_specs=[
            pl.BlockSpec((BLK, 128), lambda i: (i, 0)),
            pl.BlockSpec((BLK, 128), lambda i: (i, 0)),
            pl.BlockSpec((1, HID), lambda i: (0, 0)),
            pl.BlockSpec((8, HID), lambda i: (0, 0)),
        ],
        out_specs=[
            pl.BlockSpec((BLK, HID), lambda i: (i, 0)),
            pl.BlockSpec((2, HID), lambda i: (0, 0)),
        ],
        out_shape=[
            jax.ShapeDtypeStruct((N, HID), _f32),
            jax.ShapeDtypeStruct((2, HID), _f32),
        ],
        scratch_shapes=[pltpu.VMEM((1, HID), _f32), pltpu.VMEM((1, HID), _f32)],
    )(acc0, acc1, bg, rmat)


def _transform(x1, st, g, be, wg_t, asd):
    return _pc(
        _transform_body,
        grid=(NSTEP,),
        in_specs=[
            pl.BlockSpec((BLK, HID), lambda i: (i, 0)),
            pl.BlockSpec((2, HID), lambda i: (0, 0)),
            pl.BlockSpec((1, HID), lambda i: (0, 0)),
            pl.BlockSpec((1, HID), lambda i: (0, 0)),
            pl.BlockSpec((HID, HID), lambda i: (0, 0)),
            pl.BlockSpec((HID, 16), lambda i: (0, 0)),
        ],
        out_specs=_TAB_SPECS,
        out_shape=_TAB_SHAPES,
        scratch_shapes=[pltpu.VMEM((1, 16), _f32)],
    )(x1, st, g, be, wg_t, asd)


def _final(x1, st, g, be, w1_t, b1, w2_t, b2):
    return _pc(
        _final_body,
        grid=(NSTEP,),
        in_specs=[
            pl.BlockSpec((BLK, HID), lambda i: (i, 0)),
            pl.BlockSpec((2, HID), lambda i: (0, 0)),
            pl.BlockSpec((1, HID), lambda i: (0, 0)),
            pl.BlockSpec((1, HID), lambda i: (0, 0)),
            pl.BlockSpec((HID, HID // 2), lambda i: (0, 0)),
            pl.BlockSpec((1, HID // 2), lambda i: (0, 0)),
            pl.BlockSpec((HID // 2, 2), lambda i: (0, 0)),
            pl.BlockSpec((1, 2), lambda i: (0, 0)),
        ],
        out_specs=pl.BlockSpec((1, 2), lambda i: (0, 0)),
        out_shape=jax.ShapeDtypeStruct((1, 2), _f32),
        scratch_shapes=[pltpu.VMEM((1, HID), _f32)],
    )(x1, st, g, be, w1_t, b1, w2_t, b2)


def _dyn_gather16(x, idx):
    dn = lax.GatherDimensionNumbers(offset_dims=(), collapsed_slice_dims=(0,),
                                    start_index_map=(0,))
    return lax.gather(x, idx.reshape(16, 1), dn, (1,),
                      mode=lax.GatherScatterMode.PROMISE_IN_BOUNDS)


def _edge_sc(t_all, m16, csd, dsts):
    mesh = plsc.VectorSubcoreMesh(core_axis_name="c", subcore_axis_name="s")

    @functools.partial(
        pl.kernel,
        out_type=jax.ShapeDtypeStruct((2 * NP, 128), _f32),
        mesh=mesh,
        scratch_types=[
            pltpu.VMEM((2 * K_E,), _i32),
            pltpu.VMEM((2 * K_E,), _i32),
            pltpu.VMEM((K_E,), _i32),
            pltpu.VMEM((2 * K_E, 128), _f32),
            pltpu.VMEM((2 * K_E, 128), _f32),
            pltpu.VMEM((K_E, 128), _f32),
            pltpu.VMEM((16,), _f32),
            pltpu.VMEM_SHARED((NP, 128), _f32),
            pltpu.SemaphoreType.DMA,
            pltpu.SemaphoreType.DMA,
        ],
    )
    def k(ta_hbm, m_hbm, csd_hbm, dsts_hbm, out_hbm,
          cidxa, cidxb, didxs, rowsa, rowsb, stage, m_v, acc, sema, semb):
        cid = lax.axis_index("c")
        sid = lax.axis_index("s")
        wid = sid * 2 + cid
        row0 = sid * ROWS_PER_SUB

        @pl.loop(0, K_E)
        def _(r):
            @pl.loop(0, 128, step=16)
            def _(cc):
                stage[r, pl.ds(cc, 16)] = jnp.zeros((16,), _f32)

        @pl.loop(0, ROWS_PER_SUB, step=K_E)
        def _(rr):
            pltpu.sync_copy(stage, acc.at[pl.ds(row0 + rr, K_E)])

        pltpu.sync_copy(m_hbm, m_v)
        plsc.subcore_barrier()

        iota = lax.iota(_i32, 16)
        ib = lax.shift_right_logical(iota, 3)
        rot8 = lax.bitwise_and(iota + 8, 15)

        def fetch_idx(ck, cidx_b):
            k2 = wid * NCHUNK + ck
            pltpu.sync_copy(csd_hbm.at[pl.ds(k2 * 2 * K_E, 2 * K_E)], cidx_b)

        def compute_scatter(ck, rows_b):
            k2 = wid * NCHUNK + ck
            pltpu.sync_copy(dsts_hbm.at[pl.ds(k2 * K_E, K_E)], didxs)
            mv = m_v[...]

            @pl.loop(0, K_E)
            def _(e):
                asr = rows_b[e, pl.ds(64, 16)]
                ads = _dyn_gather16(rows_b[K_E + e, pl.ds(64, 16)], rot8)
                a = asr + ads
                a = jnp.maximum(a, 0.2 * a)
                ex = jnp.exp(a - mv)
                stage[e, pl.ds(64, 16)] = ex
                for g in range(4):
                    exb = _dyn_gather16(ex, ib + (2 * g))
                    stage[e, pl.ds(16 * g, 16)] = (
                        exb * rows_b[e, pl.ds(16 * g, 16)])

            pltpu.sync_copy(stage, acc.at[didxs], add=True)

        fetch_idx(0, cidxa)
        pltpu.async_copy(ta_hbm.at[cidxa], rowsa, sema)

        @pl.loop(0, NCHUNK, step=2)
        def _(ck):
            pltpu.make_async_copy(ta_hbm.at[cidxa], rowsa, sema).wait()
            fetch_idx(ck + 1, cidxb)
            pltpu.async_copy(ta_hbm.at[cidxb], rowsb, semb)
            compute_scatter(ck, rowsa)
            pltpu.make_async_copy(ta_hbm.at[cidxb], rowsb, semb).wait()

            @pl.when(ck + 2 < NCHUNK)
            def _():
                fetch_idx(ck + 2, cidxa)
                pltpu.async_copy(ta_hbm.at[cidxa], rowsa, sema)

            compute_scatter(ck + 1, rowsb)

        plsc.subcore_barrier()
        pltpu.sync_copy(acc.at[pl.ds(row0, ROWS_PER_SUB)],
                        out_hbm.at[pl.ds(cid * NP + row0, ROWS_PER_SUB)])

    accs = k(t_all, m16, csd, dsts)
    return accs[:NP], accs[NP:]


def _asd(a_s, a_d):
    rows = jnp.arange(64)
    heads = rows // 8
    a = jnp.zeros((64, 16), _f32)
    a = a.at[rows, heads].set(a_s.reshape(64).astype(_f32))
    a = a.at[rows, 8 + heads].set(a_d.reshape(64).astype(_f32))
    return a


def kernel(x, edge_index, W_in, b_in, Wg0, as0, ad0, bg0, g0, be0, Wg1, as1,
           ad1, bg1, g1, be1, Wg2, as2, ad2, bg2, g2, be2, W_fc1, b_fc1,
           W_fc2, b_fc2):
    loop = jnp.arange(N, dtype=edge_index.dtype)
    npad = EPAD - (E + N)
    src = jnp.concatenate(
        [edge_index[0], loop, jnp.zeros((npad,), edge_index.dtype)])
    dstg = jnp.concatenate(
        [edge_index[1], loop, jnp.zeros((npad,), edge_index.dtype)])
    dsts = jnp.concatenate(
        [edge_index[1], loop, jnp.full((npad,), N, edge_index.dtype)])
    csd = jnp.concatenate(
        [src.reshape(-1, K_E), dstg.reshape(-1, K_E)], axis=1).reshape(-1)

    rmat = jnp.repeat(jnp.eye(8, dtype=_f32), 8, axis=1)
    asds = [_asd(as0, ad0), _asd(as1, ad1), _asd(as2, ad2)]
    wgs = [Wg0.T, Wg1.T, Wg2.T]
    bgs = [bg0[None, :], bg1[None, :], bg2[None, :]]
    gs = [g0[None, :], g1[None, :], g2[None, :]]
    bes = [be0[None, :], be1[None, :], be2[None, :]]

    ta, m16 = _prelude(x, W_in.T, b_in[None, :], wgs[0], asds[0])
    for layer in range(3):
        acc0, acc1 = _edge_sc(ta, m16.reshape(16), csd, dsts)
        x1, st = _merge(acc0, acc1, bgs[layer], rmat)
        if layer < 2:
            ta, m16 = _transform(x1, st, gs[layer], bes[layer],
                                 wgs[layer + 1], asds[layer + 1])
        else:
            out = _final(x1, st, gs[layer], bes[layer], W_fc1.T,
                         b_fc1[None, :], W_fc2.T, b_fc2[None, :])
    return out
